# baseline (speedup 1.0000x reference)
import functools

import jax
import jax.numpy as jnp
from jax import lax
from jax.experimental import pallas as pl
from jax.experimental.pallas import tpu as pltpu
from jax.experimental.pallas import tpu_sc as plsc

TOPK = 8


def _qres_kernel(x_ref, qw_ref, qb_ref, rw_ref, rb_ref, q_ref, r_ref):
    q = jax.lax.dot_general(
        x_ref[...], qw_ref[...], (((1,), (1,)), ((), ())),
        preferred_element_type=jnp.float32) + qb_ref[...]
    q_ref[...] = q
    r_ref[...] = jax.lax.dot_general(
        q, rw_ref[...], (((1,), (1,)), ((), ())),
        preferred_element_type=jnp.float32) + rb_ref[...]


def _topk_kernel(q_ref, keys_ref, probs_ref, idx_ref, *, n_keys):
    keys = keys_ref[...]
    inv = jax.lax.rsqrt(jnp.sum(keys * keys, axis=1, keepdims=True))
    keys_n = keys * inv
    scores = jax.lax.dot_general(
        q_ref[...], keys_n, (((1,), (1,)), ((), ())),
        preferred_element_type=jnp.float32)
    tb = scores.shape[0]
    iota = jax.lax.broadcasted_iota(jnp.int32, (tb, n_keys), 1)
    neg_inf = jnp.float32(-jnp.inf)
    vals = []
    idxs = []
    s = scores
    for _ in range(TOPK):
        m = jnp.max(s, axis=1, keepdims=True)
        hit = s == m
        ix = jnp.min(jnp.where(hit, iota, n_keys), axis=1, keepdims=True)
        vals.append(m)
        idxs.append(ix)
        s = jnp.where(iota == ix, neg_inf, s)
    v = jnp.concatenate(vals, axis=1)
    ix = jnp.concatenate(idxs, axis=1)
    e = jnp.exp(v - v[:, 0:1])
    probs_ref[...] = e / jnp.sum(e, axis=1, keepdims=True)
    idx_ref[...] = ix + pl.program_id(0) * n_keys


def _make_bag_kernel(n_pairs, v_dim, n_slots, bs, pairs_per_w, chunk):
    vblocks = v_dim // 16

    @functools.partial(
        pl.kernel,
        mesh=plsc.VectorSubcoreMesh(core_axis_name="c", subcore_axis_name="s"),
        out_type=jax.ShapeDtypeStruct((bs, n_slots * v_dim), jnp.float32),
        scratch_types=[
            pltpu.VMEM((chunk * TOPK,), jnp.int32),
            pltpu.VMEM((chunk * TOPK + 16,), jnp.float32),
            pltpu.VMEM((chunk * TOPK, v_dim), jnp.float32),
            pltpu.VMEM((chunk, v_dim), jnp.float32),
            pltpu.SemaphoreType.DMA,
        ],
    )
    def bag(idx_hbm, probs_hbm, values_hbm, res_hbm, out_hbm,
            idx_v, probs_v, rows_v, acc_v, sem):
        cid = lax.axis_index("c")
        sid = lax.axis_index("s")
        n_cores = lax.axis_size("c")
        wid = sid * n_cores + cid
        w_per_slot = (n_pairs // pairs_per_w) // n_slots
        slot = wid // w_per_slot
        col0 = slot * v_dim

        def body(c, _):
            m0 = wid * pairs_per_w + c * chunk
            b0 = m0 - slot * bs
            pltpu.sync_copy(idx_hbm.at[pl.ds(m0 * TOPK, chunk * TOPK)], idx_v)
            pltpu.sync_copy(probs_hbm.at[pl.ds(m0 * TOPK, chunk * TOPK)],
                            probs_v.at[pl.ds(0, chunk * TOPK)])
            pltpu.async_copy(values_hbm.at[idx_v], rows_v, sem).wait()
            pltpu.sync_copy(res_hbm.at[pl.ds(b0, chunk), :], acc_v)

            def pair_body(p, _):
                row0 = p * TOPK
                pv = probs_v[pl.ds(row0, 16)]
                for vb in range(vblocks):
                    sl = pl.ds(vb * 16, 16)
                    acc = acc_v[p, sl]
                    for j in range(TOPK):
                        acc = acc + pv[j] * rows_v[row0 + j, sl]
                    acc_v[p, sl] = acc
                return 0

            lax.fori_loop(0, chunk, pair_body, 0)
            pltpu.sync_copy(acc_v,
                            out_hbm.at[pl.ds(b0, chunk), pl.ds(col0, v_dim)])
            return 0

        lax.fori_loop(0, pairs_per_w // chunk, body, 0)

    return bag


def kernel(x, query_w, query_b, binding_keys, binding_values, res_w, res_b):
    prefix = x.shape[:-1]
    d = x.shape[-1]
    bs = 1
    for p in prefix:
        bs *= p
    num_slots, n_keys, k_dim = binding_keys.shape
    v_dim = binding_values.shape[-1]
    xf = x.reshape(bs, d)

    ta = min(1024, bs)
    q, res = pl.pallas_call(
        _qres_kernel,
        grid=(bs // ta,),
        in_specs=[
            pl.BlockSpec((ta, d), lambda t: (t, 0)),
            pl.BlockSpec((k_dim, d), lambda t: (0, 0)),
            pl.BlockSpec((1, k_dim), lambda t: (0, 0)),
            pl.BlockSpec((v_dim, k_dim), lambda t: (0, 0)),
            pl.BlockSpec((1, v_dim), lambda t: (0, 0)),
        ],
        out_specs=[
            pl.BlockSpec((ta, k_dim), lambda t: (t, 0)),
            pl.BlockSpec((ta, v_dim), lambda t: (t, 0)),
        ],
        out_shape=[
            jax.ShapeDtypeStruct((bs, k_dim), jnp.float32),
            jax.ShapeDtypeStruct((bs, v_dim), jnp.float32),
        ],
    )(xf, query_w, query_b.reshape(1, k_dim), res_w, res_b.reshape(1, v_dim))

    tb = min(512, bs)
    probs, idx = pl.pallas_call(
        functools.partial(_topk_kernel, n_keys=n_keys),
        grid=(num_slots, bs // tb),
        in_specs=[
            pl.BlockSpec((tb, k_dim), lambda s, t: (t, 0)),
            pl.BlockSpec((None, n_keys, k_dim), lambda s, t: (s, 0, 0)),
        ],
        out_specs=[
            pl.BlockSpec((None, tb, TOPK), lambda s, t: (s, t, 0)),
            pl.BlockSpec((None, tb, TOPK), lambda s, t: (s, t, 0)),
        ],
        out_shape=[
            jax.ShapeDtypeStruct((num_slots, bs, TOPK), jnp.float32),
            jax.ShapeDtypeStruct((num_slots, bs, TOPK), jnp.int32),
        ],
    )(q, binding_keys)

    n_pairs = num_slots * bs
    n_workers = 32
    pairs_per_w = n_pairs // n_workers
    chunk = 16
    bag = _make_bag_kernel(n_pairs, v_dim, num_slots, bs, pairs_per_w, chunk)
    out = bag(
        idx.reshape(n_pairs * TOPK),
        probs.reshape(n_pairs * TOPK),
        binding_values.reshape(num_slots * n_keys, v_dim),
        res,
    )

    return out.reshape(prefix + (num_slots, v_dim))


# speedup vs baseline: 1.0242x; 1.0242x over previous
import functools

import jax
import jax.numpy as jnp
from jax import lax
from jax.experimental import pallas as pl
from jax.experimental.pallas import tpu as pltpu
from jax.experimental.pallas import tpu_sc as plsc

TOPK = 8


def _qres_kernel(x_ref, qw_ref, qb_ref, rw_ref, rb_ref, q_ref, r_ref):
    q = jax.lax.dot_general(
        x_ref[...], qw_ref[...], (((1,), (1,)), ((), ())),
        preferred_element_type=jnp.float32) + qb_ref[...]
    q_ref[...] = q
    r_ref[...] = jax.lax.dot_general(
        q, rw_ref[...], (((1,), (1,)), ((), ())),
        preferred_element_type=jnp.float32) + rb_ref[...]


def _topk_kernel(q_ref, keys_ref, probs_ref, idx_ref, *, n_keys):
    keys = keys_ref[...]
    inv = jax.lax.rsqrt(jnp.sum(keys * keys, axis=1, keepdims=True))
    keys_n = keys * inv
    scores = jax.lax.dot_general(
        q_ref[...], keys_n, (((1,), (1,)), ((), ())),
        preferred_element_type=jnp.float32)
    tb = scores.shape[0]
    iota = jax.lax.broadcasted_iota(jnp.int32, (tb, n_keys), 1)
    neg_inf = jnp.float32(-jnp.inf)
    vals = []
    idxs = []
    s = scores
    for _ in range(TOPK):
        m = jnp.max(s, axis=1, keepdims=True)
        hit = s == m
        ix = jnp.min(jnp.where(hit, iota, n_keys), axis=1, keepdims=True)
        vals.append(m)
        idxs.append(ix)
        s = jnp.where(hit, neg_inf, s)
    v = jnp.concatenate(vals, axis=1)
    ix = jnp.concatenate(idxs, axis=1)
    e = jnp.exp(v - v[:, 0:1])
    probs_ref[...] = e / jnp.sum(e, axis=1, keepdims=True)
    idx_ref[...] = ix + pl.program_id(0) * n_keys


def _make_bag_kernel(n_pairs, v_dim, n_slots, bs, pairs_per_w, chunk):
    vblocks = v_dim // 16

    @functools.partial(
        pl.kernel,
        mesh=plsc.VectorSubcoreMesh(core_axis_name="c", subcore_axis_name="s"),
        out_type=jax.ShapeDtypeStruct((bs, n_slots * v_dim), jnp.float32),
        scratch_types=[
            pltpu.VMEM((chunk * TOPK,), jnp.int32),
            pltpu.VMEM((chunk * TOPK + 16,), jnp.float32),
            pltpu.VMEM((chunk * TOPK, v_dim), jnp.float32),
            pltpu.VMEM((chunk, v_dim), jnp.float32),
            pltpu.SemaphoreType.DMA,
        ],
    )
    def bag(idx_hbm, probs_hbm, values_hbm, res_hbm, out_hbm,
            idx_v, probs_v, rows_v, acc_v, sem):
        cid = lax.axis_index("c")
        sid = lax.axis_index("s")
        n_cores = lax.axis_size("c")
        wid = sid * n_cores + cid
        w_per_slot = (n_pairs // pairs_per_w) // n_slots
        slot = wid // w_per_slot
        col0 = slot * v_dim

        def body(c, _):
            m0 = wid * pairs_per_w + c * chunk
            b0 = m0 - slot * bs
            pltpu.sync_copy(idx_hbm.at[pl.ds(m0 * TOPK, chunk * TOPK)], idx_v)
            pltpu.sync_copy(probs_hbm.at[pl.ds(m0 * TOPK, chunk * TOPK)],
                            probs_v.at[pl.ds(0, chunk * TOPK)])
            pltpu.async_copy(values_hbm.at[idx_v], rows_v, sem).wait()
            pltpu.sync_copy(res_hbm.at[pl.ds(b0, chunk), :], acc_v)

            def pair_body(p, _):
                row0 = p * TOPK
                pv = probs_v[pl.ds(row0, 16)]
                for vb in range(vblocks):
                    sl = pl.ds(vb * 16, 16)
                    acc = acc_v[p, sl]
                    for j in range(TOPK):
                        acc = acc + pv[j] * rows_v[row0 + j, sl]
                    acc_v[p, sl] = acc
                return 0

            lax.fori_loop(0, chunk, pair_body, 0)
            pltpu.sync_copy(acc_v,
                            out_hbm.at[pl.ds(b0, chunk), pl.ds(col0, v_dim)])
            return 0

        lax.fori_loop(0, pairs_per_w // chunk, body, 0)

    return bag


def kernel(x, query_w, query_b, binding_keys, binding_values, res_w, res_b):
    prefix = x.shape[:-1]
    d = x.shape[-1]
    bs = 1
    for p in prefix:
        bs *= p
    num_slots, n_keys, k_dim = binding_keys.shape
    v_dim = binding_values.shape[-1]
    xf = x.reshape(bs, d)

    ta = min(1024, bs)
    q, res = pl.pallas_call(
        _qres_kernel,
        grid=(bs // ta,),
        in_specs=[
            pl.BlockSpec((ta, d), lambda t: (t, 0)),
            pl.BlockSpec((k_dim, d), lambda t: (0, 0)),
            pl.BlockSpec((1, k_dim), lambda t: (0, 0)),
            pl.BlockSpec((v_dim, k_dim), lambda t: (0, 0)),
            pl.BlockSpec((1, v_dim), lambda t: (0, 0)),
        ],
        out_specs=[
            pl.BlockSpec((ta, k_dim), lambda t: (t, 0)),
            pl.BlockSpec((ta, v_dim), lambda t: (t, 0)),
        ],
        out_shape=[
            jax.ShapeDtypeStruct((bs, k_dim), jnp.float32),
            jax.ShapeDtypeStruct((bs, v_dim), jnp.float32),
        ],
    )(xf, query_w, query_b.reshape(1, k_dim), res_w, res_b.reshape(1, v_dim))

    tb = min(512, bs)
    probs, idx = pl.pallas_call(
        functools.partial(_topk_kernel, n_keys=n_keys),
        grid=(num_slots, bs // tb),
        in_specs=[
            pl.BlockSpec((tb, k_dim), lambda s, t: (t, 0)),
            pl.BlockSpec((None, n_keys, k_dim), lambda s, t: (s, 0, 0)),
        ],
        out_specs=[
            pl.BlockSpec((None, tb, TOPK), lambda s, t: (s, t, 0)),
            pl.BlockSpec((None, tb, TOPK), lambda s, t: (s, t, 0)),
        ],
        out_shape=[
            jax.ShapeDtypeStruct((num_slots, bs, TOPK), jnp.float32),
            jax.ShapeDtypeStruct((num_slots, bs, TOPK), jnp.int32),
        ],
    )(q, binding_keys)

    n_pairs = num_slots * bs
    n_workers = 32
    pairs_per_w = n_pairs // n_workers
    chunk = 16
    bag = _make_bag_kernel(n_pairs, v_dim, num_slots, bs, pairs_per_w, chunk)
    out = bag(
        idx.reshape(n_pairs * TOPK),
        probs.reshape(n_pairs * TOPK),
        binding_values.reshape(num_slots * n_keys, v_dim),
        res,
    )

    return out.reshape(prefix + (num_slots, v_dim))


# speedup vs baseline: 1.2073x; 1.1788x over previous
import functools

import jax
import jax.numpy as jnp
from jax import lax
from jax.experimental import pallas as pl
from jax.experimental.pallas import tpu as pltpu
from jax.experimental.pallas import tpu_sc as plsc

TOPK = 8


def _qres_kernel(x_ref, qw_ref, qb_ref, rw_ref, rb_ref, q_ref, r_ref):
    q = jax.lax.dot_general(
        x_ref[...], qw_ref[...], (((1,), (1,)), ((), ())),
        preferred_element_type=jnp.float32) + qb_ref[...]
    q_ref[...] = q
    r_ref[...] = jax.lax.dot_general(
        q, rw_ref[...], (((1,), (1,)), ((), ())),
        preferred_element_type=jnp.float32) + rb_ref[...]


def _topk_kernel(q_ref, keys_ref, probs_ref, idx_ref, *, n_keys, idx_base):
    keys = keys_ref[...]
    inv = jax.lax.rsqrt(jnp.sum(keys * keys, axis=1, keepdims=True))
    keys_n = keys * inv
    scores = jax.lax.dot_general(
        q_ref[...], keys_n, (((1,), (1,)), ((), ())),
        preferred_element_type=jnp.float32)
    tb = scores.shape[0]
    iota = jax.lax.broadcasted_iota(jnp.int32, (tb, n_keys), 1)
    neg_inf = jnp.float32(-jnp.inf)
    vals = []
    idxs = []
    s = scores
    for _ in range(TOPK):
        m = jnp.max(s, axis=1, keepdims=True)
        hit = s == m
        ix = jnp.min(jnp.where(hit, iota, n_keys), axis=1, keepdims=True)
        vals.append(m)
        idxs.append(ix)
        s = jnp.where(hit, neg_inf, s)
    v = jnp.concatenate(vals, axis=1)
    ix = jnp.concatenate(idxs, axis=1)
    e = jnp.exp(v - v[:, 0:1])
    probs_ref[...] = e / jnp.sum(e, axis=1, keepdims=True)
    idx_ref[...] = ix + idx_base


def _make_bag_kernel(v_dim, bs, pairs_per_w, chunk):
    vblocks = v_dim // 16
    n_chunks = pairs_per_w // chunk
    cpk = chunk * TOPK

    @functools.partial(
        pl.kernel,
        mesh=plsc.VectorSubcoreMesh(core_axis_name="c", subcore_axis_name="s"),
        out_type=jax.ShapeDtypeStruct((bs, v_dim), jnp.float32),
        scratch_types=[
            pltpu.VMEM((pairs_per_w * TOPK,), jnp.int32),
            pltpu.VMEM((pairs_per_w * TOPK + 16,), jnp.float32),
            pltpu.VMEM((cpk, v_dim), jnp.float32),
            pltpu.VMEM((cpk, v_dim), jnp.float32),
            pltpu.VMEM((chunk, v_dim), jnp.float32),
            pltpu.SemaphoreType.DMA,
            pltpu.SemaphoreType.DMA,
        ],
    )
    def bag(idx_hbm, probs_hbm, values_hbm, res_hbm, out_hbm,
            idx_v, probs_v, rows0_v, rows1_v, acc_v, sem0, sem1):
        cid = lax.axis_index("c")
        sid = lax.axis_index("s")
        n_cores = lax.axis_size("c")
        wid = sid * n_cores + cid
        p0 = wid * pairs_per_w

        pltpu.sync_copy(idx_hbm.at[pl.ds(p0 * TOPK, pairs_per_w * TOPK)],
                        idx_v)
        pltpu.sync_copy(probs_hbm.at[pl.ds(p0 * TOPK, pairs_per_w * TOPK)],
                        probs_v.at[pl.ds(0, pairs_per_w * TOPK)])

        rows = (rows0_v, rows1_v)
        sems = (sem0, sem1)

        def fire(c, buf, sem):
            pltpu.async_copy(
                values_hbm.at[idx_v.at[pl.ds(c * cpk, cpk)]], buf, sem)

        def compute(c, buf):
            b0 = p0 + c * chunk
            pltpu.sync_copy(res_hbm.at[pl.ds(b0, chunk), :], acc_v)

            def pair_body(p, _):
                row0 = p * TOPK
                pv = probs_v[pl.ds(c * cpk + row0, 16)]
                for vb in range(vblocks):
                    sl = pl.ds(vb * 16, 16)
                    acc = acc_v[p, sl]
                    for j in range(TOPK):
                        acc = acc + pv[j] * buf[row0 + j, sl]
                    acc_v[p, sl] = acc
                return 0

            lax.fori_loop(0, chunk, pair_body, 0)
            pltpu.sync_copy(acc_v, out_hbm.at[pl.ds(b0, chunk), :])

        fire(0, rows[0], sems[0])

        def body(c2, _):
            c_a = 2 * c2
            fire(c_a + 1, rows[1], sems[1])
            pltpu.make_async_copy(
                values_hbm.at[pl.ds(0, cpk), :], rows[0], sems[0]).wait()
            compute(c_a, rows[0])

            @pl.when(c_a + 2 < n_chunks)
            def _():
                fire(c_a + 2, rows[0], sems[0])

            pltpu.make_async_copy(
                values_hbm.at[pl.ds(0, cpk), :], rows[1], sems[1]).wait()
            compute(c_a + 1, rows[1])
            return 0

        lax.fori_loop(0, n_chunks // 2, body, 0)

    return bag


def kernel(x, query_w, query_b, binding_keys, binding_values, res_w, res_b):
    prefix = x.shape[:-1]
    d = x.shape[-1]
    bs = 1
    for p in prefix:
        bs *= p
    num_slots, n_keys, k_dim = binding_keys.shape
    v_dim = binding_values.shape[-1]
    xf = x.reshape(bs, d)

    ta = min(1024, bs)
    q, res = pl.pallas_call(
        _qres_kernel,
        grid=(bs // ta,),
        in_specs=[
            pl.BlockSpec((ta, d), lambda t: (t, 0)),
            pl.BlockSpec((k_dim, d), lambda t: (0, 0)),
            pl.BlockSpec((1, k_dim), lambda t: (0, 0)),
            pl.BlockSpec((v_dim, k_dim), lambda t: (0, 0)),
            pl.BlockSpec((1, v_dim), lambda t: (0, 0)),
        ],
        out_specs=[
            pl.BlockSpec((ta, k_dim), lambda t: (t, 0)),
            pl.BlockSpec((ta, v_dim), lambda t: (t, 0)),
        ],
        out_shape=[
            jax.ShapeDtypeStruct((bs, k_dim), jnp.float32),
            jax.ShapeDtypeStruct((bs, v_dim), jnp.float32),
        ],
    )(xf, query_w, query_b.reshape(1, k_dim), res_w, res_b.reshape(1, v_dim))

    values_flat = binding_values.reshape(num_slots * n_keys, v_dim)
    n_workers = 32
    pairs_per_w = bs // n_workers
    chunk = 8
    bag = _make_bag_kernel(v_dim, bs, pairs_per_w, chunk)

    tb = min(512, bs)
    outs = []
    for s in range(num_slots):
        probs_s, idx_s = pl.pallas_call(
            functools.partial(_topk_kernel, n_keys=n_keys,
                              idx_base=s * n_keys),
            grid=(bs // tb,),
            in_specs=[
                pl.BlockSpec((tb, k_dim), lambda t: (t, 0)),
                pl.BlockSpec((n_keys, k_dim), lambda t: (0, 0)),
            ],
            out_specs=[
                pl.BlockSpec((tb, TOPK), lambda t: (t, 0)),
                pl.BlockSpec((tb, TOPK), lambda t: (t, 0)),
            ],
            out_shape=[
                jax.ShapeDtypeStruct((bs, TOPK), jnp.float32),
                jax.ShapeDtypeStruct((bs, TOPK), jnp.int32),
            ],
        )(q, binding_keys[s])
        outs.append(bag(
            idx_s.reshape(bs * TOPK),
            probs_s.reshape(bs * TOPK),
            values_flat,
            res,
        ))

    out = jnp.stack(outs, axis=1)
    return out.reshape(prefix + (num_slots, v_dim))
